# Initial kernel scaffold; baseline (speedup 1.0000x reference)
import jax
import jax.numpy as jnp
from jax.experimental import pallas as pl


def kernel(x, edge_index, edge_weight, Wz, bz, Wr, br, Wh, bh, Wl, bl):
    raise NotImplementedError("write your pallas kernel here")


# speedup vs baseline: 19.8715x; 19.8715x over previous
import functools
import jax
import jax.numpy as jnp
from jax import lax
from jax.experimental import pallas as pl
from jax.experimental.pallas import tpu as pltpu
from jax.experimental.pallas import tpu_sc as plsc

N = 10000
E = 320000
D_IN = 128
D_HID = 64

NTILE = 16
CHK = 80
NCH = E // CHK
TPB = NCH // NTILE
IB = 25
ZR = N // NTILE
NBLK = 25
BR = N // NBLK


def _deg_body(eidx, wch, zn, deg_out, deg_s, idx_v, w_v):
    c = lax.axis_index("c")
    s = lax.axis_index("s")

    @pl.when(s == 0)
    def _():
        pltpu.sync_copy(zn, deg_s)

    plsc.subcore_barrier()
    pltpu.sync_copy(eidx.at[c, s], idx_v)
    pltpu.sync_copy(wch.at[s], w_v)

    @pl.loop(0, TPB)
    def _(j):
        pltpu.sync_copy(w_v.at[j], deg_s.at[idx_v.at[j]], add=True)

    plsc.subcore_barrier()

    @pl.when(s == 0)
    def _():
        pltpu.sync_copy(deg_s, deg_out.at[c])


def _diff_body(tab, gidx, sidx, zb, out, acc, gi_v, si_v, buf):
    c = lax.axis_index("c")
    s = lax.axis_index("s")
    pltpu.sync_copy(zb, acc.at[pl.ds(s * ZR, ZR)])
    plsc.subcore_barrier()

    @pl.loop(0, TPB // IB)
    def _(ib):
        pltpu.sync_copy(gidx.at[c, s, ib], gi_v)
        pltpu.sync_copy(sidx.at[c, s, ib], si_v)

        @pl.loop(0, IB)
        def _(j):
            pltpu.sync_copy(tab.at[gi_v.at[j]], buf)
            pltpu.sync_copy(buf, acc.at[si_v.at[j]], add=True)

    plsc.subcore_barrier()
    pltpu.sync_copy(acc.at[pl.ds(s * ZR, ZR)], out.at[c, s])


def _mm_body(x_ref, deg_ref, wp_ref, wq_ref, wa_ref, ba_ref, tab_ref, xa_ref):
    xb = x_ref[...]
    d = deg_ref[0]
    r_out = jnp.reciprocal(d[:, 0:1])
    r_in = jnp.reciprocal(d[:, 1:2])
    hi = lax.Precision.HIGHEST
    tab_ref[0] = jnp.dot(xb, wp_ref[...], precision=hi, preferred_element_type=jnp.float32) * r_out
    tab_ref[1] = jnp.dot(xb, wq_ref[...], precision=hi, preferred_element_type=jnp.float32) * r_in
    xa_ref[...] = jnp.dot(xb, wa_ref[...], precision=hi, preferred_element_type=jnp.float32) + ba_ref[...]


def _fin_body(xa_ref, s_ref, wl_ref, bl_ref, o_ref):
    xa = xa_ref[...]
    s_o = s_ref[0]
    s_i = s_ref[1]
    zp = xa[:, :D_HID] + s_o[:, :D_HID] + s_i[:, :D_HID]
    hp = xa[:, D_HID:] + s_o[:, D_HID:] + s_i[:, D_HID:]
    hnew = (1.0 - jax.nn.sigmoid(zp)) * jnp.tanh(hp)
    o_ref[...] = jnp.dot(hnew, wl_ref[...], precision=lax.Precision.HIGHEST, preferred_element_type=jnp.float32) + bl_ref[...]


def kernel(x, edge_index, edge_weight, Wz, bz, Wr, br, Wh, bh, Wl, bl):
    f32 = jnp.float32
    ei = edge_index.astype(jnp.int32)
    row, col = ei[0], ei[1]
    eidx = ei.reshape(2, NTILE, TPB, CHK)
    gidx = jnp.stack([row, col + N]).reshape(2, NTILE, TPB // IB, IB, CHK)
    sidx = jnp.stack([col, row]).reshape(2, NTILE, TPB // IB, IB, CHK)
    wch = edge_weight.astype(f32).reshape(NTILE, TPB, CHK)
    zn = jnp.zeros((N,), f32)
    zb = jnp.zeros((ZR, D_IN), f32)

    W_P = jnp.concatenate([Wz[0, 1, :D_IN, :], Wh[0, 1, :D_IN, :]], axis=1)
    W_Q = jnp.concatenate([Wz[1, 1, :D_IN, :], Wh[1, 1, :D_IN, :]], axis=1)
    W_A = jnp.concatenate(
        [Wz[0, 0, :D_IN] + Wz[1, 0, :D_IN], Wh[0, 0, :D_IN] + Wh[1, 0, :D_IN]], axis=1
    )
    b_A = jnp.concatenate([bz, bh]).reshape(1, 2 * D_HID)

    mesh = plsc.VectorSubcoreMesh(core_axis_name="c", subcore_axis_name="s")

    deg = pl.kernel(
        _deg_body,
        out_type=jax.ShapeDtypeStruct((2, N), f32),
        mesh=mesh,
        scratch_types=[
            pltpu.VMEM_SHARED((N,), f32),
            pltpu.VMEM((TPB, CHK), jnp.int32),
            pltpu.VMEM((TPB, CHK), f32),
        ],
        name="dcrnn_degrees",
    )(eidx, wch, zn)
    deg_t = deg.T.reshape(NBLK, BR, 2)

    tab3, xa = pl.pallas_call(
        _mm_body,
        grid=(NBLK,),
        in_specs=[
            pl.BlockSpec((BR, D_IN), lambda i: (i, 0)),
            pl.BlockSpec((1, BR, 2), lambda i: (i, 0, 0)),
            pl.BlockSpec((D_IN, D_IN), lambda i: (0, 0)),
            pl.BlockSpec((D_IN, D_IN), lambda i: (0, 0)),
            pl.BlockSpec((D_IN, D_IN), lambda i: (0, 0)),
            pl.BlockSpec((1, 2 * D_HID), lambda i: (0, 0)),
        ],
        out_specs=[
            pl.BlockSpec((2, BR, D_IN), lambda i: (0, i, 0)),
            pl.BlockSpec((BR, D_IN), lambda i: (i, 0)),
        ],
        out_shape=[
            jax.ShapeDtypeStruct((2, N, D_IN), f32),
            jax.ShapeDtypeStruct((N, D_IN), f32),
        ],
        name="dcrnn_tables",
    )(x, deg_t, W_P, W_Q, W_A, b_A)

    tab = tab3.reshape(2 * N, D_IN)

    s_acc = pl.kernel(
        _diff_body,
        out_type=jax.ShapeDtypeStruct((2, NTILE, ZR, D_IN), f32),
        mesh=mesh,
        scratch_types=[
            pltpu.VMEM_SHARED((N, D_IN), f32),
            pltpu.VMEM((IB, CHK), jnp.int32),
            pltpu.VMEM((IB, CHK), jnp.int32),
            pltpu.VMEM((CHK, D_IN), f32),
        ],
        name="dcrnn_diffusion",
    )(tab, gidx, sidx, zb)
    s_acc = s_acc.reshape(2, N, D_IN)

    out = pl.pallas_call(
        _fin_body,
        grid=(NBLK,),
        in_specs=[
            pl.BlockSpec((BR, D_IN), lambda i: (i, 0)),
            pl.BlockSpec((2, BR, D_IN), lambda i: (0, i, 0)),
            pl.BlockSpec((D_HID, 1), lambda i: (0, 0)),
            pl.BlockSpec((1, 1), lambda i: (0, 0)),
        ],
        out_specs=pl.BlockSpec((BR, 1), lambda i: (i, 0)),
        out_shape=jax.ShapeDtypeStruct((N, 1), f32),
        name="dcrnn_head",
    )(xa, s_acc, Wl, bl.reshape(1, 1))

    return out[:, 0]
